# Initial kernel scaffold; baseline (speedup 1.0000x reference)
import jax
import jax.numpy as jnp
from jax.experimental import pallas as pl


def kernel(x, edge_index, batch_idx, W1, a_src1, a_dst1, b1, W2, a_src2, a_dst2, b2, W3, a_src3, a_dst3, b3, W_ih1, W_hh1, b_ih1, b_hh1, W_ih2, W_hh2, b_ih2, b_hh2, Wo, bo):
    raise NotImplementedError("write your pallas kernel here")


# speedup vs baseline: 1.0168x; 1.0168x over previous
import functools

import jax
import jax.numpy as jnp
from jax.experimental import pallas as pl

N_NODES = 10000
N_EDGES = 160000
N_GRAPHS = 64
HEADS = 8
H1, H2, H3 = 128, 256, 512
GRU_H = 256


def _mm_body(a_ref, w_ref, o_ref):
    o_ref[...] = jnp.dot(a_ref[...], w_ref[...],
                         preferred_element_type=jnp.float32)


def _matmul(a, w, bn=1000, bf=512):
    n, k = a.shape
    k2, f = w.shape
    assert k == k2 and n % bn == 0 and f % bf == 0
    return pl.pallas_call(
        _mm_body,
        grid=(n // bn, f // bf),
        in_specs=[
            pl.BlockSpec((bn, k), lambda i, j: (i, 0)),
            pl.BlockSpec((k, bf), lambda i, j: (0, j)),
        ],
        out_specs=pl.BlockSpec((bn, bf), lambda i, j: (i, j)),
        out_shape=jax.ShapeDtypeStruct((n, f), jnp.float32),
    )(a, w)


def _gat(x, src, dst, W, a_src, a_dst, bias, heads, out_dim, concat):
    n = x.shape[0]
    k = x.shape[1]
    if k % 128 != 0:
        kp = ((k + 127) // 128) * 128
        x = jnp.pad(x, ((0, 0), (0, kp - k)))
        W = jnp.pad(W, ((0, kp - k), (0, 0)))
    h = _matmul(x, W).reshape(n, heads, out_dim)
    al_src = (h * a_src[None, :, :]).sum(-1)
    al_dst = (h * a_dst[None, :, :]).sum(-1)
    e = al_src[src] + al_dst[dst]
    e = jnp.where(e > 0, e, 0.2 * e)
    e_max = jax.lax.stop_gradient(jax.ops.segment_max(e, dst, num_segments=n))
    e_max = jnp.where(jnp.isfinite(e_max), e_max, 0.0)
    ex = jnp.exp(e - e_max[dst])
    denom = jax.ops.segment_sum(ex, dst, num_segments=n)
    alpha = ex / (denom[dst] + 1e-16)
    msg = h[src] * alpha[:, :, None]
    out = jax.ops.segment_sum(msg, dst, num_segments=n)
    if concat:
        return out.reshape(n, heads * out_dim) + bias
    return out.mean(axis=1) + bias


def _gru_cell(x, h, W_ih, W_hh, b_ih, b_hh):
    gi = x @ W_ih.T + b_ih
    gh = h @ W_hh.T + b_hh
    i_r, i_z, i_n = jnp.split(gi, 3, axis=-1)
    h_r, h_z, h_n = jnp.split(gh, 3, axis=-1)
    r = jax.nn.sigmoid(i_r + h_r)
    z = jax.nn.sigmoid(i_z + h_z)
    ncand = jnp.tanh(i_n + r * h_n)
    return (1.0 - z) * ncand + z * h


def kernel(x, edge_index, batch_idx, W1, a_src1, a_dst1, b1, W2, a_src2,
           a_dst2, b2, W3, a_src3, a_dst3, b3, W_ih1, W_hh1, b_ih1, b_hh1,
           W_ih2, W_hh2, b_ih2, b_hh2, Wo, bo):
    src, dst = edge_index[0], edge_index[1]
    h = jax.nn.relu(_gat(x, src, dst, W1, a_src1, a_dst1, b1, HEADS, H1, True))
    h = jax.nn.relu(_gat(h, src, dst, W2, a_src2, a_dst2, b2, HEADS, H2, True))
    h = jax.nn.relu(_gat(h, src, dst, W3, a_src3, a_dst3, b3, 1, H3, False))
    counts = jax.ops.segment_sum(jnp.ones((h.shape[0],), dtype=h.dtype),
                                 batch_idx, num_segments=N_GRAPHS)
    graph_emb = (jax.ops.segment_sum(h, batch_idx, num_segments=N_GRAPHS)
                 / jnp.clip(counts, 1.0)[:, None])
    h0 = jnp.zeros((N_GRAPHS, GRU_H), dtype=h.dtype)
    h1 = _gru_cell(graph_emb, h0, W_ih1, W_hh1, b_ih1, b_hh1)
    h2 = _gru_cell(h1, h0, W_ih2, W_hh2, b_ih2, b_hh2)
    return h2 @ Wo.T + bo


# speedup vs baseline: 7.0720x; 6.9552x over previous
import functools

import jax
import jax.numpy as jnp
from jax import lax
from jax.experimental import pallas as pl
from jax.experimental.pallas import tpu as pltpu
from jax.experimental.pallas import tpu_sc as plsc

_INTERPRET = False

N_NODES = 10000
NP = 10240
N_EDGES = 160000
N_GRAPHS = 64
HEADS = 8
H1, H2, H3 = 128, 256, 512
GRU_H = 256
BN = 1280
NT = NP // BN


def _ka_body(a_ref, w_ref, bias_ref, avec_ref, hc_ref, al_ref, m_ref, macc,
             *, relu_in, c_in, c_out, dpc):
    c = pl.program_id(1)
    i = pl.program_id(0)
    acc = jnp.zeros((BN, 128), jnp.float32)
    for kc in range(c_in):
        ab = a_ref[kc]
        if relu_in:
            ab = jnp.maximum(ab + bias_ref[kc, 0:1, :], 0.0)
        acc += jnp.dot(ab, w_ref[kc], preferred_element_type=jnp.float32)
    hc_ref[0] = acc

    asrc = avec_ref[0, 0, 0:1, :]
    adst = avec_ref[1, 0, 0:1, :]
    psrc = jnp.sum(acc * asrc, axis=1, keepdims=True)
    pdst = jnp.sum(acc * adst, axis=1, keepdims=True)
    hd = c // dpc
    lane = lax.broadcasted_iota(jnp.int32, (BN, 128), 1)
    contrib = (jnp.where(lane == hd, psrc, 0.0)
               + jnp.where(lane == 16 + hd, pdst, 0.0))

    @pl.when(c == 0)
    def _():
        al_ref[...] = contrib

    @pl.when(c > 0)
    def _():
        al_ref[...] = al_ref[...] + contrib

    @pl.when(c == c_out - 1)
    def _():
        bm = jnp.broadcast_to(jnp.max(al_ref[...], axis=0, keepdims=True),
                              (8, 128))

        @pl.when(i == 0)
        def _():
            macc[...] = bm

        @pl.when(i > 0)
        def _():
            macc[...] = jnp.maximum(macc[...], bm)

        @pl.when(i == NT - 1)
        def _():
            m_ref[...] = macc[...]


def _stage_a(a, w_r, bias_c, avec, relu_in, dpc):
    c_in = a.shape[0]
    c_out = w_r.shape[2] // 128
    body = functools.partial(_ka_body, relu_in=relu_in, c_in=c_in,
                             c_out=c_out, dpc=dpc)
    return pl.pallas_call(
        body,
        grid=(NT, c_out),
        in_specs=[
            pl.BlockSpec((c_in, BN, 128), lambda i, c: (0, i, 0)),
            pl.BlockSpec((c_in, 128, 128), lambda i, c: (0, 0, c)),
            pl.BlockSpec((c_in, 8, 128), lambda i, c: (0, 0, 0)),
            pl.BlockSpec((2, 1, 8, 128), lambda i, c: (0, c, 0, 0)),
        ],
        out_specs=[
            pl.BlockSpec((1, BN, 128), lambda i, c: (c, i, 0)),
            pl.BlockSpec((BN, 128), lambda i, c: (i, 0)),
            pl.BlockSpec((8, 128), lambda i, c: (0, 0)),
        ],
        out_shape=[
            jax.ShapeDtypeStruct((c_out, NP, 128), jnp.float32),
            jax.ShapeDtypeStruct((NP, 128), jnp.float32),
            jax.ShapeDtypeStruct((8, 128), jnp.float32),
        ],
        scratch_shapes=[pltpu.VMEM((8, 128), jnp.float32)],
        interpret=_INTERPRET,
    )(a, w_r, bias_c, avec)


NB = N_EDGES // 128
_SC_MESH = dict(core_axis_name="c", subcore_axis_name="s")
ROWS_PER_TEC = NP // 16


def _b1_body(src_ref, dst_ref, al_ref, m_ref, p_ref, d_ref,
             sidx, didx, arows, brows, pbuf, mbuf, zbuf, acc, sem):
    cid = lax.axis_index("c")
    sid = lax.axis_index("s")
    w = sid * 2 + cid

    @pl.loop(0, ROWS_PER_TEC)
    def _(r):
        zbuf[r, :] = jnp.zeros((16,), jnp.float32)

    pltpu.sync_copy(zbuf, acc.at[pl.ds(sid * ROWS_PER_TEC, ROWS_PER_TEC)])
    plsc.subcore_barrier()

    pltpu.sync_copy(m_ref.at[pl.ds(0, 1)], mbuf)
    msum = mbuf[0, 0:16] + mbuf[0, 16:32]
    mv = jnp.where(msum > 0, msum, 0.2 * msum)

    @pl.loop(0, 40)
    def _(b):
        gb = b * 32 + w

        @pl.when(gb < NB)
        def _():
            off = gb * 128
            pltpu.sync_copy(src_ref.at[pl.ds(off, 128)], sidx)
            pltpu.sync_copy(dst_ref.at[pl.ds(off, 128)], didx)
            pltpu.async_copy(al_ref.at[sidx], arows, sem).wait()
            pltpu.async_copy(al_ref.at[didx], brows, sem).wait()

            @pl.loop(0, 128)
            def _(j):
                sv = arows[j, 0:16] + brows[j, 16:32]
                ev = jnp.where(sv > 0, sv, 0.2 * sv)
                pbuf[j, :] = jnp.exp(ev - mv)

            pltpu.sync_copy(pbuf, p_ref.at[pl.ds(off, 128)])
            pltpu.sync_copy(pbuf, acc.at[didx], add=True)

    plsc.subcore_barrier()
    pltpu.sync_copy(acc.at[pl.ds(sid * ROWS_PER_TEC, ROWS_PER_TEC)], zbuf)
    pltpu.sync_copy(
        zbuf, d_ref.at[cid].at[pl.ds(sid * ROWS_PER_TEC, ROWS_PER_TEC)])


def _b2_body(src_ref, dst_ref, p_ref, d_ref, w_ref,
             didx, pbuf, d0, d1, wbuf, sem):
    del src_ref
    cid = lax.axis_index("c")
    sid = lax.axis_index("s")
    w = sid * 2 + cid

    @pl.loop(0, 40)
    def _(b):
        gb = b * 32 + w

        @pl.when(gb < NB)
        def _():
            off = gb * 128
            pltpu.sync_copy(dst_ref.at[pl.ds(off, 128)], didx)
            pltpu.sync_copy(p_ref.at[pl.ds(off, 128)], pbuf)
            pltpu.async_copy(d_ref.at[0].at[didx], d0, sem).wait()
            pltpu.async_copy(d_ref.at[1].at[didx], d1, sem).wait()

            @pl.loop(0, 128)
            def _(j):
                dsum = d0[j, :] + d1[j, :] + 1e-16
                wbuf[j, :] = pbuf[j, :] / dsum

            pltpu.sync_copy(wbuf, w_ref.at[pl.ds(off, 128)])


def _c_body(hc_ref, src_ref, dst_ref, w_ref, out_ref,
            sidx, didx, wbuf, rows, obuf, acc, sem, *, c_out, dpc):
    cid = lax.axis_index("c")
    sid = lax.axis_index("s")

    cpc = c_out // 2
    for k in range(cpc):
        ci = cid * cpc + k
        hd16 = jnp.full((16,), ci // dpc, jnp.int32)

        @pl.loop(0, 128)
        def _(r):
            for q in range(8):
                obuf.at[r][16 * q:16 * q + 16] = jnp.zeros((16,),
                                                           jnp.float32)

        for t in range(5):
            pltpu.sync_copy(
                obuf, acc.at[pl.ds(sid * ROWS_PER_TEC + t * 128, 128)])
        plsc.subcore_barrier()

        @pl.loop(0, 79)
        def _(b):
            gb = b * 16 + sid

            @pl.when(gb < NB)
            def _():
                off = gb * 128
                pltpu.sync_copy(src_ref.at[pl.ds(off, 128)], sidx)
                pltpu.sync_copy(dst_ref.at[pl.ds(off, 128)], didx)
                pltpu.sync_copy(w_ref.at[pl.ds(off, 128)], wbuf)
                pltpu.async_copy(hc_ref.at[ci].at[sidx], rows, sem).wait()

                @pl.loop(0, 128)
                def _(j):
                    wv = plsc.load_gather(
                        wbuf, [jnp.full((16,), j, jnp.int32), hd16])
                    for q in range(8):
                        sl = slice(16 * q, 16 * q + 16)
                        rows.at[j][sl] = rows.at[j][sl] * wv

                pltpu.sync_copy(rows, acc.at[didx], add=True)

        plsc.subcore_barrier()
        for t in range(5):
            ro = sid * ROWS_PER_TEC + t * 128
            pltpu.sync_copy(acc.at[pl.ds(ro, 128)], obuf)
            pltpu.sync_copy(obuf, out_ref.at[ci].at[pl.ds(ro, 128)])
        plsc.subcore_barrier()


def _edge_softmax(al, m_arr, src, dst):
    mesh = plsc.VectorSubcoreMesh(**_SC_MESH)
    p, d = pl.kernel(
        _b1_body,
        out_type=[jax.ShapeDtypeStruct((N_EDGES, 16), jnp.float32),
                  jax.ShapeDtypeStruct((2, NP, 16), jnp.float32)],
        mesh=mesh,
        scratch_types=[
            pltpu.VMEM((128,), jnp.int32),
            pltpu.VMEM((128,), jnp.int32),
            pltpu.VMEM((128, 128), jnp.float32),
            pltpu.VMEM((128, 128), jnp.float32),
            pltpu.VMEM((128, 16), jnp.float32),
            pltpu.VMEM((1, 128), jnp.float32),
            pltpu.VMEM((ROWS_PER_TEC, 16), jnp.float32),
            pltpu.VMEM_SHARED((NP, 16), jnp.float32),
            pltpu.SemaphoreType.DMA,
        ],
        compiler_params=pltpu.CompilerParams(use_tc_tiling_on_sc=False),
        name="gat_edge_logits",
    )(src, dst, al, m_arr)

    w_att = pl.kernel(
        _b2_body,
        out_type=jax.ShapeDtypeStruct((N_EDGES, 16), jnp.float32),
        mesh=mesh,
        scratch_types=[
            pltpu.VMEM((128,), jnp.int32),
            pltpu.VMEM((128, 16), jnp.float32),
            pltpu.VMEM((128, 16), jnp.float32),
            pltpu.VMEM((128, 16), jnp.float32),
            pltpu.VMEM((128, 16), jnp.float32),
            pltpu.SemaphoreType.DMA,
        ],
        compiler_params=pltpu.CompilerParams(use_tc_tiling_on_sc=False),
        name="gat_edge_norm",
    )(src, dst, p, d)
    return w_att


def _aggregate(hc, w_att, src, dst, dpc):
    c_out = hc.shape[0]
    mesh = plsc.VectorSubcoreMesh(**_SC_MESH)
    body = functools.partial(_c_body, c_out=c_out, dpc=dpc)
    return pl.kernel(
        body,
        out_type=jax.ShapeDtypeStruct((c_out, NP, 128), jnp.float32),
        mesh=mesh,
        scratch_types=[
            pltpu.VMEM((128,), jnp.int32),
            pltpu.VMEM((128,), jnp.int32),
            pltpu.VMEM((128, 16), jnp.float32),
            pltpu.VMEM((128, 128), jnp.float32),
            pltpu.VMEM((128, 128), jnp.float32),
            pltpu.VMEM_SHARED((NP, 128), jnp.float32),
            pltpu.SemaphoreType.DMA,
        ],
        compiler_params=pltpu.CompilerParams(use_tc_tiling_on_sc=False,
                                             needs_layout_passes=False),
        name="gat_aggregate",
    )(hc, src, dst, w_att)


def _kd_body(hc_ref, bidx_ref, b3_ref, wih1_ref, bih1_ref, bhh1_ref,
             wih2_ref, bih2_ref, bhh2_ref, wo_ref, bo_ref, out_ref,
             emb_acc, cnt_acc):
    i = pl.program_id(0)

    @pl.when(i == 0)
    def _():
        emb_acc[...] = jnp.zeros_like(emb_acc)
        cnt_acc[...] = jnp.zeros_like(cnt_acc)

    bidx = bidx_ref[0]
    onehot = (lax.broadcasted_iota(jnp.int32, (N_GRAPHS, BN), 0)
              == bidx).astype(jnp.float32)
    for kc in range(4):
        h3c = jnp.maximum(hc_ref[kc] + b3_ref[kc, 0:1, :], 0.0)
        emb_acc[:, kc * 128:(kc + 1) * 128] += jnp.dot(
            onehot, h3c, preferred_element_type=jnp.float32)
    cnt_acc[...] += jnp.broadcast_to(
        jnp.sum(onehot, axis=1, keepdims=True), cnt_acc.shape)

    @pl.when(i == NT - 1)
    def _():
        cnt = jnp.maximum(cnt_acc[:, 0:1], 1.0)
        emb = emb_acc[...] / cnt

        def sig(v):
            return 1.0 / (1.0 + jnp.exp(-v))

        gi1 = lax.dot_general(emb, wih1_ref[...],
                              (((1,), (1,)), ((), ()))) + bih1_ref[0:1, :]
        bh1 = bhh1_ref[0:1, :]
        r1 = sig(gi1[:, :256] + bh1[:, :256])
        z1 = sig(gi1[:, 256:512] + bh1[:, 256:512])
        n1 = jnp.tanh(gi1[:, 512:768] + r1 * bh1[:, 512:768])
        h1 = (1.0 - z1) * n1

        gi2 = lax.dot_general(h1, wih2_ref[...],
                              (((1,), (1,)), ((), ()))) + bih2_ref[0:1, :]
        bh2 = bhh2_ref[0:1, :]
        r2 = sig(gi2[:, :256] + bh2[:, :256])
        z2 = sig(gi2[:, 256:512] + bh2[:, 256:512])
        n2 = jnp.tanh(gi2[:, 512:768] + r2 * bh2[:, 512:768])
        h2 = (1.0 - z2) * n2

        out_ref[...] = lax.dot_general(
            h2, wo_ref[...], (((1,), (1,)), ((), ()))) + bo_ref[0:1, :]


def _stage_d(hc3, bidx3d, b3c, W_ih1, b_ih1, b_hh1, W_ih2, b_ih2, b_hh2,
             Wo, bo):
    full = lambda shape: pl.BlockSpec(shape, lambda i: tuple(
        0 for _ in shape))
    return pl.pallas_call(
        _kd_body,
        grid=(NT,),
        in_specs=[
            pl.BlockSpec((4, BN, 128), lambda i: (0, i, 0)),
            pl.BlockSpec((1, 1, BN), lambda i: (i, 0, 0)),
            full((4, 8, 128)),
            full((3 * GRU_H, H3)),
            full((8, 3 * GRU_H)),
            full((8, 3 * GRU_H)),
            full((3 * GRU_H, GRU_H)),
            full((8, 3 * GRU_H)),
            full((8, 3 * GRU_H)),
            full((H3, GRU_H)),
            full((8, H3)),
        ],
        out_specs=pl.BlockSpec((N_GRAPHS, H3), lambda i: (0, 0)),
        out_shape=jax.ShapeDtypeStruct((N_GRAPHS, H3), jnp.float32),
        scratch_shapes=[pltpu.VMEM((N_GRAPHS, H3), jnp.float32),
                        pltpu.VMEM((N_GRAPHS, 128), jnp.float32)],
        interpret=_INTERPRET,
    )(hc3, bidx3d, b3c, W_ih1, b_ih1, b_hh1, W_ih2, b_ih2, b_hh2, Wo, bo)


def _chunk_vecs(a_src, a_dst, c_out):
    asrc_c = a_src.reshape(c_out, 128)
    adst_c = a_dst.reshape(c_out, 128)
    av = jnp.stack([asrc_c, adst_c])
    return jnp.broadcast_to(av[:, :, None, :], (2, c_out, 8, 128))


def _bias_chunks(b, c):
    return jnp.broadcast_to(b.reshape(c, 1, 128), (c, 8, 128))


def _layer(a, src, dst, W, a_src, a_dst, b_prev, relu_in, heads, out_dim):
    c_in = a.shape[0]
    c_out = heads * out_dim // 128
    dpc = out_dim // 128
    w_r = W.reshape(c_in, 128, heads * out_dim)
    bias_c = (_bias_chunks(b_prev, c_in) if relu_in
              else jnp.zeros((c_in, 8, 128), jnp.float32))
    avec = _chunk_vecs(a_src, a_dst, c_out)
    hc, al, m_arr = _stage_a(a, w_r, bias_c, avec, relu_in, dpc)
    w_att = _edge_softmax(al, m_arr, src, dst)
    return _aggregate(hc, w_att, src, dst, dpc)


def kernel(x, edge_index, batch_idx, W1, a_src1, a_dst1, b1, W2, a_src2,
           a_dst2, b2, W3, a_src3, a_dst3, b3, W_ih1, W_hh1, b_ih1, b_hh1,
           W_ih2, W_hh2, b_ih2, b_hh2, Wo, bo):
    src, dst = edge_index[0], edge_index[1]
    xp = jnp.pad(x, ((0, NP - N_NODES), (0, 128 - x.shape[1])))[None]
    w1p = jnp.pad(W1, ((0, 128 - W1.shape[0]), (0, 0)))

    g1 = _layer(xp, src, dst, w1p, a_src1, a_dst1, None, False, HEADS, H1)
    g2 = _layer(g1, src, dst, W2, a_src2, a_dst2, b1, True, HEADS, H2)
    g3 = _layer(g2, src, dst, W3, a_src3, a_dst3, b2, True, 1, H3)

    bidx3d = jnp.pad(batch_idx, (0, NP - N_NODES),
                     constant_values=N_GRAPHS).reshape(NT, 1, BN)
    b3c = _bias_chunks(b3, 4)
    b8 = lambda v: jnp.broadcast_to(v[None, :], (8, v.shape[0]))
    return _stage_d(g3, bidx3d, b3c, W_ih1, b8(b_ih1), b8(b_hh1), W_ih2,
                    b8(b_ih2), b8(b_hh2), Wo, b8(bo))


# speedup vs baseline: 12.5717x; 1.7777x over previous
import functools

import jax
import jax.numpy as jnp
from jax import lax
from jax.experimental import pallas as pl
from jax.experimental.pallas import tpu as pltpu
from jax.experimental.pallas import tpu_sc as plsc

_INTERPRET = False

N_NODES = 10000
NP = 10240
N_EDGES = 160000
N_GRAPHS = 64
HEADS = 8
H1, H2, H3 = 128, 256, 512
GRU_H = 256
BN = 1280
NT = NP // BN


def _ka_body(a_ref, w_ref, bias_ref, avec_ref, hc_ref, al_ref, m_ref, macc,
             *, relu_in, c_in, c_out, dpc):
    c = pl.program_id(1)
    i = pl.program_id(0)
    acc = jnp.zeros((BN, 128), jnp.float32)
    for kc in range(c_in):
        ab = a_ref[kc]
        if relu_in:
            ab = jnp.maximum(ab + bias_ref[kc, 0:1, :], 0.0)
        acc += jnp.dot(ab, w_ref[kc], preferred_element_type=jnp.float32)
    hc_ref[0] = acc

    asrc = avec_ref[0, 0, 0:1, :]
    adst = avec_ref[1, 0, 0:1, :]
    psrc = jnp.sum(acc * asrc, axis=1, keepdims=True)
    pdst = jnp.sum(acc * adst, axis=1, keepdims=True)
    hd = c // dpc
    lane = lax.broadcasted_iota(jnp.int32, (BN, 128), 1)
    contrib = (jnp.where(lane == hd, psrc, 0.0)
               + jnp.where(lane == 16 + hd, pdst, 0.0))

    @pl.when(c == 0)
    def _():
        al_ref[...] = contrib

    @pl.when(c > 0)
    def _():
        al_ref[...] = al_ref[...] + contrib

    @pl.when(c == c_out - 1)
    def _():
        bm = jnp.broadcast_to(jnp.max(al_ref[...], axis=0, keepdims=True),
                              (8, 128))

        @pl.when(i == 0)
        def _():
            macc[...] = bm

        @pl.when(i > 0)
        def _():
            macc[...] = jnp.maximum(macc[...], bm)

        @pl.when(i == NT - 1)
        def _():
            m_ref[...] = macc[...]


def _stage_a(a, w_r, bias_c, avec, relu_in, dpc):
    c_in = a.shape[0]
    c_out = w_r.shape[2] // 128
    body = functools.partial(_ka_body, relu_in=relu_in, c_in=c_in,
                             c_out=c_out, dpc=dpc)
    return pl.pallas_call(
        body,
        grid=(NT, c_out),
        in_specs=[
            pl.BlockSpec((c_in, BN, 128), lambda i, c: (0, i, 0)),
            pl.BlockSpec((c_in, 128, 128), lambda i, c: (0, 0, c)),
            pl.BlockSpec((c_in, 8, 128), lambda i, c: (0, 0, 0)),
            pl.BlockSpec((2, 1, 8, 128), lambda i, c: (0, c, 0, 0)),
        ],
        out_specs=[
            pl.BlockSpec((1, BN, 128), lambda i, c: (c, i, 0)),
            pl.BlockSpec((BN, 128), lambda i, c: (i, 0)),
            pl.BlockSpec((8, 128), lambda i, c: (0, 0)),
        ],
        out_shape=[
            jax.ShapeDtypeStruct((c_out, NP, 128), jnp.float32),
            jax.ShapeDtypeStruct((NP, 128), jnp.float32),
            jax.ShapeDtypeStruct((8, 128), jnp.float32),
        ],
        scratch_shapes=[pltpu.VMEM((8, 128), jnp.float32)],
        interpret=_INTERPRET,
    )(a, w_r, bias_c, avec)


NB = N_EDGES // 128
_SC_MESH = dict(core_axis_name="c", subcore_axis_name="s")
ROWS_PER_TEC = NP // 16


def _repack_body(al_ref, als_ref, ald_ref, buf, sbuf, dbuf):
    cid = lax.axis_index("c")
    sid = lax.axis_index("s")
    w = sid * 2 + cid
    base = w * (NP // 32)
    pltpu.sync_copy(al_ref.at[pl.ds(base, NP // 32)], buf)

    @pl.loop(0, NP // 32)
    def _(r):
        sbuf[r, :] = buf[r, 0:16]
        dbuf[r, :] = buf[r, 16:32]

    pltpu.sync_copy(sbuf, als_ref.at[pl.ds(base, NP // 32)])
    pltpu.sync_copy(dbuf, ald_ref.at[pl.ds(base, NP // 32)])


def _repack(al):
    mesh = plsc.VectorSubcoreMesh(**_SC_MESH)
    return pl.kernel(
        _repack_body,
        out_type=[jax.ShapeDtypeStruct((NP, 16), jnp.float32),
                  jax.ShapeDtypeStruct((NP, 16), jnp.float32)],
        mesh=mesh,
        scratch_types=[
            pltpu.VMEM((NP // 32, 128), jnp.float32),
            pltpu.VMEM((NP // 32, 16), jnp.float32),
            pltpu.VMEM((NP // 32, 16), jnp.float32),
        ],
        compiler_params=pltpu.CompilerParams(use_tc_tiling_on_sc=False),
        name="gat_repack",
    )(al)


def _b1_body(src_ref, dst_ref, als_ref, ald_ref, m_ref, p_ref, d_ref,
             sidxb, didxb, ar0, ar1, br0, br1, pb0, pb1, mbuf, zbuf, acc,
             gsem, ssem):
    cid = lax.axis_index("c")
    sid = lax.axis_index("s")
    w = sid * 2 + cid
    EB1 = 40
    NB1 = 125
    AR = (ar0, ar1)
    BR = (br0, br1)
    PB = (pb0, pb1)

    @pl.loop(0, ROWS_PER_TEC)
    def _(r):
        zbuf[r, :] = jnp.zeros((16,), jnp.float32)

    pltpu.sync_copy(zbuf, acc.at[pl.ds(sid * ROWS_PER_TEC, ROWS_PER_TEC)])

    pltpu.sync_copy(src_ref.at[pl.ds(w * NB1, NB1)], sidxb)
    pltpu.sync_copy(dst_ref.at[pl.ds(w * NB1, NB1)], didxb)
    pltpu.sync_copy(m_ref.at[pl.ds(0, 1)], mbuf)
    msum = mbuf[0, 0:16] + mbuf[0, 16:32]
    mv = jnp.where(msum > 0, msum, 0.2 * msum)
    plsc.subcore_barrier()

    @pl.loop(0, 64)
    def _(t):
        for k2 in range(2):
            b = t * 2 + k2
            s_a, s_b, s_p = AR[k2], BR[k2], PB[k2]
            p_a, p_b2, p_p = AR[1 - k2], BR[1 - k2], PB[1 - k2]

            @pl.when(jnp.logical_and(b >= 3, b <= NB1 + 2))
            def _():
                pltpu.make_async_copy(
                    als_ref.at[pl.ds(0, EB1)], s_p, ssem).wait()

            @pl.when(b < NB1)
            def _():
                pltpu.async_copy(als_ref.at[sidxb.at[b]], s_a, gsem)
                pltpu.async_copy(ald_ref.at[didxb.at[b]], s_b, gsem)

            @pl.when(jnp.logical_and(b >= 1, b <= NB1))
            def _():
                pltpu.make_async_copy(
                    als_ref.at[sidxb.at[b - 1]], p_a, gsem).wait()
                pltpu.make_async_copy(
                    ald_ref.at[didxb.at[b - 1]], p_b2, gsem).wait()

                @pl.loop(0, EB1, unroll=4)
                def _(j):
                    sv = p_a[j, :] + p_b2[j, :]
                    ev = jnp.where(sv > 0, sv, 0.2 * sv)
                    p_p[j, :] = jnp.exp(ev - mv)

                off = (w * NB1 + b - 1) * EB1
                pltpu.sync_copy(p_p, p_ref.at[pl.ds(off, EB1)])
                pltpu.async_copy(p_p, acc.at[didxb.at[b - 1]], ssem,
                                 add=True)

    plsc.subcore_barrier()
    pltpu.sync_copy(acc.at[pl.ds(sid * ROWS_PER_TEC, ROWS_PER_TEC)], zbuf)
    pltpu.sync_copy(
        zbuf, d_ref.at[cid].at[pl.ds(sid * ROWS_PER_TEC, ROWS_PER_TEC)])


def _b2_body(src_ref, dst_ref, p_ref, d_ref, w_ref,
             didx, pbuf, d0, d1, wbuf, sem):
    del src_ref
    cid = lax.axis_index("c")
    sid = lax.axis_index("s")
    w = sid * 2 + cid

    @pl.loop(0, 40)
    def _(b):
        gb = b * 32 + w

        @pl.when(gb < NB)
        def _():
            off = gb * 128
            pltpu.sync_copy(dst_ref.at[pl.ds(off, 128)], didx)
            pltpu.sync_copy(p_ref.at[pl.ds(off, 128)], pbuf)
            pltpu.async_copy(d_ref.at[0].at[didx], d0, sem).wait()
            pltpu.async_copy(d_ref.at[1].at[didx], d1, sem).wait()

            @pl.loop(0, 128)
            def _(j):
                dsum = d0[j, :] + d1[j, :] + 1e-16
                wbuf[j, :] = pbuf[j, :] / dsum

            pltpu.sync_copy(wbuf, w_ref.at[pl.ds(off, 128)])


def _c_body(hc_ref, src_ref, dst_ref, w_ref, out_ref,
            sidxb, didxb, wb0, wb1, wb2, rows0, rows1, rows2,
            acc, gsem, ssem, wsem, *, c_out, dpc):
    cid = lax.axis_index("c")
    sid = lax.axis_index("s")
    WB = (wb0, wb1, wb2)
    RW = (rows0, rows1, rows2)
    EB = 40
    NBT = 250

    pltpu.sync_copy(src_ref.at[pl.ds(sid * NBT, NBT)], sidxb)
    pltpu.sync_copy(dst_ref.at[pl.ds(sid * NBT, NBT)], didxb)

    cpc = c_out // 2
    for k in range(cpc):
        ci = cid * cpc + k
        hd16 = jnp.full((16,), ci // dpc, jnp.int32)

        @pl.loop(0, EB)
        def _(r):
            for q in range(8):
                rows0.at[r][16 * q:16 * q + 16] = jnp.zeros((16,),
                                                            jnp.float32)

        for t in range(16):
            pltpu.sync_copy(
                rows0, acc.at[pl.ds(sid * ROWS_PER_TEC + t * EB, EB)])
        plsc.subcore_barrier()

        @pl.loop(0, 84)
        def _(tt):
            for k3 in range(3):
                b = tt * 3 + k3
                s_w, s_r = WB[k3], RW[k3]
                p_w, p_r = WB[(k3 + 2) % 3], RW[(k3 + 2) % 3]

                @pl.when(jnp.logical_and(b >= 2, b <= NBT + 1))
                def _():
                    pltpu.make_async_copy(
                        hc_ref.at[0].at[pl.ds(0, EB)], s_r, ssem).wait()

                @pl.when(b < NBT)
                def _():
                    pltpu.async_copy(
                        w_ref.at[pl.ds((sid * NBT + b) * EB, EB)], s_w,
                        wsem)
                    pltpu.async_copy(
                        hc_ref.at[ci].at[sidxb.at[b]], s_r, gsem)

                @pl.when(jnp.logical_and(b >= 1, b <= NBT))
                def _():
                    pltpu.make_async_copy(
                        hc_ref.at[ci].at[sidxb.at[b - 1]], p_r,
                        gsem).wait()
                    pltpu.make_async_copy(
                        w_ref.at[pl.ds(0, EB)], p_w, wsem).wait()

                    @pl.loop(0, EB, unroll=4)
                    def _(j):
                        wv = plsc.load_gather(
                            p_w, [jnp.full((16,), j, jnp.int32), hd16])
                        for q in range(8):
                            sl = slice(16 * q, 16 * q + 16)
                            p_r.at[j][sl] = p_r.at[j][sl] * wv

                    pltpu.async_copy(p_r, acc.at[didxb.at[b - 1]], ssem,
                                     add=True)

        plsc.subcore_barrier()
        for t in range(16):
            ro = sid * ROWS_PER_TEC + t * EB
            pltpu.sync_copy(acc.at[pl.ds(ro, EB)], rows1)
            pltpu.sync_copy(rows1, out_ref.at[ci].at[pl.ds(ro, EB)])
        plsc.subcore_barrier()


def _edge_softmax(al, m_arr, src, dst):
    als, ald = _repack(al)
    mesh = plsc.VectorSubcoreMesh(**_SC_MESH)
    src2d = src.reshape(N_EDGES // 40, 40)
    dst2d = dst.reshape(N_EDGES // 40, 40)
    p, d = pl.kernel(
        _b1_body,
        out_type=[jax.ShapeDtypeStruct((N_EDGES, 16), jnp.float32),
                  jax.ShapeDtypeStruct((2, NP, 16), jnp.float32)],
        mesh=mesh,
        scratch_types=[
            pltpu.VMEM((125, 40), jnp.int32),
            pltpu.VMEM((125, 40), jnp.int32),
            pltpu.VMEM((40, 16), jnp.float32),
            pltpu.VMEM((40, 16), jnp.float32),
            pltpu.VMEM((40, 16), jnp.float32),
            pltpu.VMEM((40, 16), jnp.float32),
            pltpu.VMEM((40, 16), jnp.float32),
            pltpu.VMEM((40, 16), jnp.float32),
            pltpu.VMEM((1, 128), jnp.float32),
            pltpu.VMEM((ROWS_PER_TEC, 16), jnp.float32),
            pltpu.VMEM_SHARED((NP, 16), jnp.float32),
            pltpu.SemaphoreType.DMA,
            pltpu.SemaphoreType.DMA,
        ],
        compiler_params=pltpu.CompilerParams(use_tc_tiling_on_sc=False),
        name="gat_edge_logits",
    )(src2d, dst2d, als, ald, m_arr)

    w_att = pl.kernel(
        _b2_body,
        out_type=jax.ShapeDtypeStruct((N_EDGES, 16), jnp.float32),
        mesh=mesh,
        scratch_types=[
            pltpu.VMEM((128,), jnp.int32),
            pltpu.VMEM((128, 16), jnp.float32),
            pltpu.VMEM((128, 16), jnp.float32),
            pltpu.VMEM((128, 16), jnp.float32),
            pltpu.VMEM((128, 16), jnp.float32),
            pltpu.SemaphoreType.DMA,
        ],
        compiler_params=pltpu.CompilerParams(use_tc_tiling_on_sc=False),
        name="gat_edge_norm",
    )(src, dst, p, d)
    return w_att


def _aggregate(hc, w_att, src, dst, dpc):
    c_out = hc.shape[0]
    mesh = plsc.VectorSubcoreMesh(**_SC_MESH)
    body = functools.partial(_c_body, c_out=c_out, dpc=dpc)
    src2d = src.reshape(N_EDGES // 40, 40)
    dst2d = dst.reshape(N_EDGES // 40, 40)
    return pl.kernel(
        body,
        out_type=jax.ShapeDtypeStruct((c_out, NP, 128), jnp.float32),
        mesh=mesh,
        scratch_types=[
            pltpu.VMEM((250, 40), jnp.int32),
            pltpu.VMEM((250, 40), jnp.int32),
            pltpu.VMEM((40, 16), jnp.float32),
            pltpu.VMEM((40, 16), jnp.float32),
            pltpu.VMEM((40, 16), jnp.float32),
            pltpu.VMEM((40, 128), jnp.float32),
            pltpu.VMEM((40, 128), jnp.float32),
            pltpu.VMEM((40, 128), jnp.float32),
            pltpu.VMEM_SHARED((NP, 128), jnp.float32),
            pltpu.SemaphoreType.DMA,
            pltpu.SemaphoreType.DMA,
            pltpu.SemaphoreType.DMA,
        ],
        compiler_params=pltpu.CompilerParams(use_tc_tiling_on_sc=False,
                                             needs_layout_passes=False),
        name="gat_aggregate",
    )(hc, src2d, dst2d, w_att)


def _kd_body(hc_ref, bidx_ref, b3_ref, wih1_ref, bih1_ref, bhh1_ref,
             wih2_ref, bih2_ref, bhh2_ref, wo_ref, bo_ref, out_ref,
             emb_acc, cnt_acc):
    i = pl.program_id(0)

    @pl.when(i == 0)
    def _():
        emb_acc[...] = jnp.zeros_like(emb_acc)
        cnt_acc[...] = jnp.zeros_like(cnt_acc)

    bidx = bidx_ref[0]
    onehot = (lax.broadcasted_iota(jnp.int32, (N_GRAPHS, BN), 0)
              == bidx).astype(jnp.float32)
    for kc in range(4):
        h3c = jnp.maximum(hc_ref[kc] + b3_ref[kc, 0:1, :], 0.0)
        emb_acc[:, kc * 128:(kc + 1) * 128] += jnp.dot(
            onehot, h3c, preferred_element_type=jnp.float32)
    cnt_acc[...] += jnp.broadcast_to(
        jnp.sum(onehot, axis=1, keepdims=True), cnt_acc.shape)

    @pl.when(i == NT - 1)
    def _():
        cnt = jnp.maximum(cnt_acc[:, 0:1], 1.0)
        emb = emb_acc[...] / cnt

        def sig(v):
            return 1.0 / (1.0 + jnp.exp(-v))

        gi1 = lax.dot_general(emb, wih1_ref[...],
                              (((1,), (1,)), ((), ()))) + bih1_ref[0:1, :]
        bh1 = bhh1_ref[0:1, :]
        r1 = sig(gi1[:, :256] + bh1[:, :256])
        z1 = sig(gi1[:, 256:512] + bh1[:, 256:512])
        n1 = jnp.tanh(gi1[:, 512:768] + r1 * bh1[:, 512:768])
        h1 = (1.0 - z1) * n1

        gi2 = lax.dot_general(h1, wih2_ref[...],
                              (((1,), (1,)), ((), ()))) + bih2_ref[0:1, :]
        bh2 = bhh2_ref[0:1, :]
        r2 = sig(gi2[:, :256] + bh2[:, :256])
        z2 = sig(gi2[:, 256:512] + bh2[:, 256:512])
        n2 = jnp.tanh(gi2[:, 512:768] + r2 * bh2[:, 512:768])
        h2 = (1.0 - z2) * n2

        out_ref[...] = lax.dot_general(
            h2, wo_ref[...], (((1,), (1,)), ((), ()))) + bo_ref[0:1, :]


def _stage_d(hc3, bidx3d, b3c, W_ih1, b_ih1, b_hh1, W_ih2, b_ih2, b_hh2,
             Wo, bo):
    full = lambda shape: pl.BlockSpec(shape, lambda i: tuple(
        0 for _ in shape))
    return pl.pallas_call(
        _kd_body,
        grid=(NT,),
        in_specs=[
            pl.BlockSpec((4, BN, 128), lambda i: (0, i, 0)),
            pl.BlockSpec((1, 1, BN), lambda i: (i, 0, 0)),
            full((4, 8, 128)),
            full((3 * GRU_H, H3)),
            full((8, 3 * GRU_H)),
            full((8, 3 * GRU_H)),
            full((3 * GRU_H, GRU_H)),
            full((8, 3 * GRU_H)),
            full((8, 3 * GRU_H)),
            full((H3, GRU_H)),
            full((8, H3)),
        ],
        out_specs=pl.BlockSpec((N_GRAPHS, H3), lambda i: (0, 0)),
        out_shape=jax.ShapeDtypeStruct((N_GRAPHS, H3), jnp.float32),
        scratch_shapes=[pltpu.VMEM((N_GRAPHS, H3), jnp.float32),
                        pltpu.VMEM((N_GRAPHS, 128), jnp.float32)],
        interpret=_INTERPRET,
    )(hc3, bidx3d, b3c, W_ih1, b_ih1, b_hh1, W_ih2, b_ih2, b_hh2, Wo, bo)


def _chunk_vecs(a_src, a_dst, c_out):
    asrc_c = a_src.reshape(c_out, 128)
    adst_c = a_dst.reshape(c_out, 128)
    av = jnp.stack([asrc_c, adst_c])
    return jnp.broadcast_to(av[:, :, None, :], (2, c_out, 8, 128))


def _bias_chunks(b, c):
    return jnp.broadcast_to(b.reshape(c, 1, 128), (c, 8, 128))


def _layer(a, src, dst, W, a_src, a_dst, b_prev, relu_in, heads, out_dim):
    c_in = a.shape[0]
    c_out = heads * out_dim // 128
    dpc = out_dim // 128
    w_r = W.reshape(c_in, 128, heads * out_dim)
    bias_c = (_bias_chunks(b_prev, c_in) if relu_in
              else jnp.zeros((c_in, 8, 128), jnp.float32))
    avec = _chunk_vecs(a_src, a_dst, c_out)
    hc, al, m_arr = _stage_a(a, w_r, bias_c, avec, relu_in, dpc)
    w_att = _edge_softmax(al, m_arr, src, dst)
    return _aggregate(hc, w_att, src, dst, dpc)


def kernel(x, edge_index, batch_idx, W1, a_src1, a_dst1, b1, W2, a_src2,
           a_dst2, b2, W3, a_src3, a_dst3, b3, W_ih1, W_hh1, b_ih1, b_hh1,
           W_ih2, W_hh2, b_ih2, b_hh2, Wo, bo):
    src, dst = edge_index[0], edge_index[1]
    xp = jnp.pad(x, ((0, NP - N_NODES), (0, 128 - x.shape[1])))[None]
    w1p = jnp.pad(W1, ((0, 128 - W1.shape[0]), (0, 0)))

    g1 = _layer(xp, src, dst, w1p, a_src1, a_dst1, None, False, HEADS, H1)
    g2 = _layer(g1, src, dst, W2, a_src2, a_dst2, b1, True, HEADS, H2)
    g3 = _layer(g2, src, dst, W3, a_src3, a_dst3, b2, True, 1, H3)

    bidx3d = jnp.pad(batch_idx, (0, NP - N_NODES),
                     constant_values=N_GRAPHS).reshape(NT, 1, BN)
    b3c = _bias_chunks(b3, 4)
    b8 = lambda v: jnp.broadcast_to(v[None, :], (8, v.shape[0]))
    return _stage_d(g3, bidx3d, b3c, W_ih1, b8(b_ih1), b8(b_hh1), W_ih2,
                    b8(b_ih2), b8(b_hh2), Wo, b8(bo))


# speedup vs baseline: 12.7247x; 1.0122x over previous
import functools

import jax
import jax.numpy as jnp
from jax import lax
from jax.experimental import pallas as pl
from jax.experimental.pallas import tpu as pltpu
from jax.experimental.pallas import tpu_sc as plsc

_INTERPRET = False

N_NODES = 10000
NP = 10240
N_EDGES = 160000
N_GRAPHS = 64
HEADS = 8
H1, H2, H3 = 128, 256, 512
GRU_H = 256
BN = 1280
NT = NP // BN


def _ka_body(a_ref, w_ref, bias_ref, avec_ref, hc_ref, al_ref, m_ref, macc,
             *, relu_in, c_in, c_out, dpc):
    c = pl.program_id(1)
    i = pl.program_id(0)
    acc = jnp.zeros((BN, 128), jnp.float32)
    for kc in range(c_in):
        ab = a_ref[kc]
        if relu_in:
            ab = jnp.maximum(ab + bias_ref[kc, 0:1, :], 0.0)
        acc += jnp.dot(ab, w_ref[kc], preferred_element_type=jnp.float32)
    hc_ref[0] = acc

    asrc = avec_ref[0, 0, 0:1, :]
    adst = avec_ref[1, 0, 0:1, :]
    psrc = jnp.sum(acc * asrc, axis=1, keepdims=True)
    pdst = jnp.sum(acc * adst, axis=1, keepdims=True)
    hd = c // dpc
    lane = lax.broadcasted_iota(jnp.int32, (BN, 128), 1)
    contrib = (jnp.where(lane == hd, psrc, 0.0)
               + jnp.where(lane == 16 + hd, pdst, 0.0))

    @pl.when(c == 0)
    def _():
        al_ref[...] = contrib

    @pl.when(c > 0)
    def _():
        al_ref[...] = al_ref[...] + contrib

    @pl.when(c == c_out - 1)
    def _():
        bm = jnp.broadcast_to(jnp.max(al_ref[...], axis=0, keepdims=True),
                              (8, 128))

        @pl.when(i == 0)
        def _():
            macc[...] = bm

        @pl.when(i > 0)
        def _():
            macc[...] = jnp.maximum(macc[...], bm)

        @pl.when(i == NT - 1)
        def _():
            m_ref[...] = macc[...]


def _stage_a(a, w_r, bias_c, avec, relu_in, dpc):
    c_in = a.shape[0]
    c_out = w_r.shape[2] // 128
    body = functools.partial(_ka_body, relu_in=relu_in, c_in=c_in,
                             c_out=c_out, dpc=dpc)
    return pl.pallas_call(
        body,
        grid=(NT, c_out),
        in_specs=[
            pl.BlockSpec((c_in, BN, 128), lambda i, c: (0, i, 0)),
            pl.BlockSpec((c_in, 128, 128), lambda i, c: (0, 0, c)),
            pl.BlockSpec((c_in, 8, 128), lambda i, c: (0, 0, 0)),
            pl.BlockSpec((2, 1, 8, 128), lambda i, c: (0, c, 0, 0)),
        ],
        out_specs=[
            pl.BlockSpec((1, BN, 128), lambda i, c: (c, i, 0)),
            pl.BlockSpec((BN, 128), lambda i, c: (i, 0)),
            pl.BlockSpec((8, 128), lambda i, c: (0, 0)),
        ],
        out_shape=[
            jax.ShapeDtypeStruct((c_out, NP, 128), jnp.float32),
            jax.ShapeDtypeStruct((NP, 128), jnp.float32),
            jax.ShapeDtypeStruct((8, 128), jnp.float32),
        ],
        scratch_shapes=[pltpu.VMEM((8, 128), jnp.float32)],
        interpret=_INTERPRET,
    )(a, w_r, bias_c, avec)


NB = N_EDGES // 128
_SC_MESH = dict(core_axis_name="c", subcore_axis_name="s")
ROWS_PER_TEC = NP // 16


def _repack_body(al_ref, als_ref, ald_ref, buf, sbuf, dbuf):
    cid = lax.axis_index("c")
    sid = lax.axis_index("s")
    w = sid * 2 + cid
    base = w * (NP // 32)
    pltpu.sync_copy(al_ref.at[pl.ds(base, NP // 32)], buf)

    @pl.loop(0, NP // 32)
    def _(r):
        sbuf[r, :] = buf[r, 0:16]
        dbuf[r, :] = buf[r, 16:32]

    pltpu.sync_copy(sbuf, als_ref.at[pl.ds(base, NP // 32)])
    pltpu.sync_copy(dbuf, ald_ref.at[pl.ds(base, NP // 32)])


def _repack(al):
    mesh = plsc.VectorSubcoreMesh(**_SC_MESH)
    return pl.kernel(
        _repack_body,
        out_type=[jax.ShapeDtypeStruct((NP, 16), jnp.float32),
                  jax.ShapeDtypeStruct((NP, 16), jnp.float32)],
        mesh=mesh,
        scratch_types=[
            pltpu.VMEM((NP // 32, 128), jnp.float32),
            pltpu.VMEM((NP // 32, 16), jnp.float32),
            pltpu.VMEM((NP // 32, 16), jnp.float32),
        ],
        compiler_params=pltpu.CompilerParams(use_tc_tiling_on_sc=False),
        name="gat_repack",
    )(al)


def _b1_body(src_ref, dst_ref, als_ref, ald_ref, m_ref, p_ref, d_ref,
             sidxb, didxb, ar0, ar1, br0, br1, pb0, pb1, mbuf, zbuf, acc,
             gsem, ssem):
    cid = lax.axis_index("c")
    sid = lax.axis_index("s")
    w = sid * 2 + cid
    EB1 = 40
    NB1 = 125
    AR = (ar0, ar1)
    BR = (br0, br1)
    PB = (pb0, pb1)

    @pl.loop(0, ROWS_PER_TEC)
    def _(r):
        zbuf[r, :] = jnp.zeros((16,), jnp.float32)

    pltpu.sync_copy(zbuf, acc.at[pl.ds(sid * ROWS_PER_TEC, ROWS_PER_TEC)])

    pltpu.sync_copy(src_ref.at[pl.ds(w * NB1, NB1)], sidxb)
    pltpu.sync_copy(dst_ref.at[pl.ds(w * NB1, NB1)], didxb)
    pltpu.sync_copy(m_ref.at[pl.ds(0, 1)], mbuf)
    msum = mbuf[0, 0:16] + mbuf[0, 16:32]
    mv = jnp.where(msum > 0, msum, 0.2 * msum)
    plsc.subcore_barrier()

    @pl.loop(0, 64)
    def _(t):
        for k2 in range(2):
            b = t * 2 + k2
            s_a, s_b, s_p = AR[k2], BR[k2], PB[k2]
            p_a, p_b2, p_p = AR[1 - k2], BR[1 - k2], PB[1 - k2]

            @pl.when(jnp.logical_and(b >= 3, b <= NB1 + 2))
            def _():
                pltpu.make_async_copy(
                    als_ref.at[pl.ds(0, EB1)], s_p, ssem).wait()

            @pl.when(b < NB1)
            def _():
                pltpu.async_copy(als_ref.at[sidxb.at[b]], s_a, gsem)
                pltpu.async_copy(ald_ref.at[didxb.at[b]], s_b, gsem)

            @pl.when(jnp.logical_and(b >= 1, b <= NB1))
            def _():
                pltpu.make_async_copy(
                    als_ref.at[sidxb.at[b - 1]], p_a, gsem).wait()
                pltpu.make_async_copy(
                    ald_ref.at[didxb.at[b - 1]], p_b2, gsem).wait()

                @pl.loop(0, EB1, unroll=4)
                def _(j):
                    sv = p_a[j, :] + p_b2[j, :]
                    ev = jnp.where(sv > 0, sv, 0.2 * sv)
                    p_p[j, :] = jnp.exp(ev - mv)

                off = (w * NB1 + b - 1) * EB1
                pltpu.sync_copy(p_p, p_ref.at[pl.ds(off, EB1)])
                pltpu.async_copy(p_p, acc.at[didxb.at[b - 1]], ssem,
                                 add=True)

    plsc.subcore_barrier()
    pltpu.sync_copy(acc.at[pl.ds(sid * ROWS_PER_TEC, ROWS_PER_TEC)], zbuf)
    pltpu.sync_copy(
        zbuf, d_ref.at[cid].at[pl.ds(sid * ROWS_PER_TEC, ROWS_PER_TEC)])


def _b2_body(src_ref, dst_ref, p_ref, d_ref, w_ref,
             didxb, d00, d01, d10, d11, pb0, pb1, wb0, wb1, gsem, psem):
    del src_ref
    cid = lax.axis_index("c")
    sid = lax.axis_index("s")
    w = sid * 2 + cid
    EB1 = 40
    NB1 = 125
    D0 = (d00, d01)
    D1 = (d10, d11)
    PB = (pb0, pb1)
    WBUF = (wb0, wb1)

    pltpu.sync_copy(dst_ref.at[pl.ds(w * NB1, NB1)], didxb)

    @pl.loop(0, 64)
    def _(t):
        for k2 in range(2):
            b = t * 2 + k2
            s_d0, s_d1, s_p = D0[k2], D1[k2], PB[k2]
            p_d0, p_d1, p_p, p_w = (D0[1 - k2], D1[1 - k2], PB[1 - k2],
                                    WBUF[1 - k2])

            @pl.when(b < NB1)
            def _():
                pltpu.async_copy(d_ref.at[0].at[didxb.at[b]], s_d0, gsem)
                pltpu.async_copy(d_ref.at[1].at[didxb.at[b]], s_d1, gsem)
                pltpu.async_copy(
                    p_ref.at[pl.ds((w * NB1 + b) * EB1, EB1)], s_p, psem)

            @pl.when(jnp.logical_and(b >= 1, b <= NB1))
            def _():
                pltpu.make_async_copy(
                    d_ref.at[0].at[didxb.at[b - 1]], p_d0, gsem).wait()
                pltpu.make_async_copy(
                    d_ref.at[0].at[didxb.at[b - 1]], p_d1, gsem).wait()
                pltpu.make_async_copy(
                    p_ref.at[pl.ds(0, EB1)], p_p, psem).wait()

                @pl.loop(0, EB1, unroll=4)
                def _(j):
                    dsum = p_d0[j, :] + p_d1[j, :] + 1e-16
                    p_w[j, :] = p_p[j, :] / dsum

                pltpu.sync_copy(
                    p_w, w_ref.at[pl.ds((w * NB1 + b - 1) * EB1, EB1)])


def _c_body(hc_ref, src_ref, dst_ref, w_ref, out_ref,
            sidxb, didxb, wb0, wb1, wb2, rows0, rows1, rows2,
            acc, gsem, ssem, wsem, *, c_out, dpc):
    cid = lax.axis_index("c")
    sid = lax.axis_index("s")
    WB = (wb0, wb1, wb2)
    RW = (rows0, rows1, rows2)
    EB = 40
    NBT = 250

    pltpu.sync_copy(src_ref.at[pl.ds(sid * NBT, NBT)], sidxb)
    pltpu.sync_copy(dst_ref.at[pl.ds(sid * NBT, NBT)], didxb)

    cpc = c_out // 2
    for k in range(cpc):
        ci = cid * cpc + k
        hd16 = jnp.full((16,), ci // dpc, jnp.int32)

        @pl.loop(0, EB)
        def _(r):
            for q in range(8):
                rows0.at[r][16 * q:16 * q + 16] = jnp.zeros((16,),
                                                            jnp.float32)

        for t in range(16):
            pltpu.sync_copy(
                rows0, acc.at[pl.ds(sid * ROWS_PER_TEC + t * EB, EB)])
        plsc.subcore_barrier()

        @pl.loop(0, 84)
        def _(tt):
            for k3 in range(3):
                b = tt * 3 + k3
                s_w, s_r = WB[k3], RW[k3]
                p_w, p_r = WB[(k3 + 2) % 3], RW[(k3 + 2) % 3]

                @pl.when(jnp.logical_and(b >= 2, b <= NBT + 1))
                def _():
                    pltpu.make_async_copy(
                        hc_ref.at[0].at[pl.ds(0, EB)], s_r, ssem).wait()

                @pl.when(b < NBT)
                def _():
                    pltpu.async_copy(
                        w_ref.at[pl.ds((sid * NBT + b) * EB, EB)], s_w,
                        wsem)
                    pltpu.async_copy(
                        hc_ref.at[ci].at[sidxb.at[b]], s_r, gsem)

                @pl.when(jnp.logical_and(b >= 1, b <= NBT))
                def _():
                    pltpu.make_async_copy(
                        hc_ref.at[ci].at[sidxb.at[b - 1]], p_r,
                        gsem).wait()
                    pltpu.make_async_copy(
                        w_ref.at[pl.ds(0, EB)], p_w, wsem).wait()

                    @pl.loop(0, EB, unroll=4)
                    def _(j):
                        wv = plsc.load_gather(
                            p_w, [jnp.full((16,), j, jnp.int32), hd16])
                        for q in range(8):
                            sl = slice(16 * q, 16 * q + 16)
                            p_r.at[j][sl] = p_r.at[j][sl] * wv

                    pltpu.async_copy(p_r, acc.at[didxb.at[b - 1]], ssem,
                                     add=True)

        plsc.subcore_barrier()
        for t in range(16):
            ro = sid * ROWS_PER_TEC + t * EB
            pltpu.sync_copy(acc.at[pl.ds(ro, EB)], rows1)
            pltpu.sync_copy(rows1, out_ref.at[ci].at[pl.ds(ro, EB)])
        plsc.subcore_barrier()


def _edge_softmax(al, m_arr, src, dst):
    als, ald = _repack(al)
    mesh = plsc.VectorSubcoreMesh(**_SC_MESH)
    src2d = src.reshape(N_EDGES // 40, 40)
    dst2d = dst.reshape(N_EDGES // 40, 40)
    p, d = pl.kernel(
        _b1_body,
        out_type=[jax.ShapeDtypeStruct((N_EDGES, 16), jnp.float32),
                  jax.ShapeDtypeStruct((2, NP, 16), jnp.float32)],
        mesh=mesh,
        scratch_types=[
            pltpu.VMEM((125, 40), jnp.int32),
            pltpu.VMEM((125, 40), jnp.int32),
            pltpu.VMEM((40, 16), jnp.float32),
            pltpu.VMEM((40, 16), jnp.float32),
            pltpu.VMEM((40, 16), jnp.float32),
            pltpu.VMEM((40, 16), jnp.float32),
            pltpu.VMEM((40, 16), jnp.float32),
            pltpu.VMEM((40, 16), jnp.float32),
            pltpu.VMEM((1, 128), jnp.float32),
            pltpu.VMEM((ROWS_PER_TEC, 16), jnp.float32),
            pltpu.VMEM_SHARED((NP, 16), jnp.float32),
            pltpu.SemaphoreType.DMA,
            pltpu.SemaphoreType.DMA,
        ],
        compiler_params=pltpu.CompilerParams(use_tc_tiling_on_sc=False),
        name="gat_edge_logits",
    )(src2d, dst2d, als, ald, m_arr)

    w_att = pl.kernel(
        _b2_body,
        out_type=jax.ShapeDtypeStruct((N_EDGES, 16), jnp.float32),
        mesh=mesh,
        scratch_types=[
            pltpu.VMEM((125, 40), jnp.int32),
            pltpu.VMEM((40, 16), jnp.float32),
            pltpu.VMEM((40, 16), jnp.float32),
            pltpu.VMEM((40, 16), jnp.float32),
            pltpu.VMEM((40, 16), jnp.float32),
            pltpu.VMEM((40, 16), jnp.float32),
            pltpu.VMEM((40, 16), jnp.float32),
            pltpu.VMEM((40, 16), jnp.float32),
            pltpu.VMEM((40, 16), jnp.float32),
            pltpu.SemaphoreType.DMA,
            pltpu.SemaphoreType.DMA,
        ],
        compiler_params=pltpu.CompilerParams(use_tc_tiling_on_sc=False),
        name="gat_edge_norm",
    )(src2d, dst2d, p, d)
    return w_att


def _aggregate(hc, w_att, src, dst, dpc):
    c_out = hc.shape[0]
    mesh = plsc.VectorSubcoreMesh(**_SC_MESH)
    body = functools.partial(_c_body, c_out=c_out, dpc=dpc)
    src2d = src.reshape(N_EDGES // 40, 40)
    dst2d = dst.reshape(N_EDGES // 40, 40)
    return pl.kernel(
        body,
        out_type=jax.ShapeDtypeStruct((c_out, NP, 128), jnp.float32),
        mesh=mesh,
        scratch_types=[
            pltpu.VMEM((250, 40), jnp.int32),
            pltpu.VMEM((250, 40), jnp.int32),
            pltpu.VMEM((40, 16), jnp.float32),
            pltpu.VMEM((40, 16), jnp.float32),
            pltpu.VMEM((40, 16), jnp.float32),
            pltpu.VMEM((40, 128), jnp.float32),
            pltpu.VMEM((40, 128), jnp.float32),
            pltpu.VMEM((40, 128), jnp.float32),
            pltpu.VMEM_SHARED((NP, 128), jnp.float32),
            pltpu.SemaphoreType.DMA,
            pltpu.SemaphoreType.DMA,
            pltpu.SemaphoreType.DMA,
        ],
        compiler_params=pltpu.CompilerParams(use_tc_tiling_on_sc=False,
                                             needs_layout_passes=False),
        name="gat_aggregate",
    )(hc, src2d, dst2d, w_att)


def _kd_body(hc_ref, bidx_ref, b3_ref, wih1_ref, bih1_ref, bhh1_ref,
             wih2_ref, bih2_ref, bhh2_ref, wo_ref, bo_ref, out_ref,
             emb_acc, cnt_acc):
    i = pl.program_id(0)

    @pl.when(i == 0)
    def _():
        emb_acc[...] = jnp.zeros_like(emb_acc)
        cnt_acc[...] = jnp.zeros_like(cnt_acc)

    bidx = bidx_ref[0]
    onehot = (lax.broadcasted_iota(jnp.int32, (N_GRAPHS, BN), 0)
              == bidx).astype(jnp.float32)
    for kc in range(4):
        h3c = jnp.maximum(hc_ref[kc] + b3_ref[kc, 0:1, :], 0.0)
        emb_acc[:, kc * 128:(kc + 1) * 128] += jnp.dot(
            onehot, h3c, preferred_element_type=jnp.float32)
    cnt_acc[...] += jnp.broadcast_to(
        jnp.sum(onehot, axis=1, keepdims=True), cnt_acc.shape)

    @pl.when(i == NT - 1)
    def _():
        cnt = jnp.maximum(cnt_acc[:, 0:1], 1.0)
        emb = emb_acc[...] / cnt

        def sig(v):
            return 1.0 / (1.0 + jnp.exp(-v))

        gi1 = lax.dot_general(emb, wih1_ref[...],
                              (((1,), (1,)), ((), ()))) + bih1_ref[0:1, :]
        bh1 = bhh1_ref[0:1, :]
        r1 = sig(gi1[:, :256] + bh1[:, :256])
        z1 = sig(gi1[:, 256:512] + bh1[:, 256:512])
        n1 = jnp.tanh(gi1[:, 512:768] + r1 * bh1[:, 512:768])
        h1 = (1.0 - z1) * n1

        gi2 = lax.dot_general(h1, wih2_ref[...],
                              (((1,), (1,)), ((), ()))) + bih2_ref[0:1, :]
        bh2 = bhh2_ref[0:1, :]
        r2 = sig(gi2[:, :256] + bh2[:, :256])
        z2 = sig(gi2[:, 256:512] + bh2[:, 256:512])
        n2 = jnp.tanh(gi2[:, 512:768] + r2 * bh2[:, 512:768])
        h2 = (1.0 - z2) * n2

        out_ref[...] = lax.dot_general(
            h2, wo_ref[...], (((1,), (1,)), ((), ()))) + bo_ref[0:1, :]


def _stage_d(hc3, bidx3d, b3c, W_ih1, b_ih1, b_hh1, W_ih2, b_ih2, b_hh2,
             Wo, bo):
    full = lambda shape: pl.BlockSpec(shape, lambda i: tuple(
        0 for _ in shape))
    return pl.pallas_call(
        _kd_body,
        grid=(NT,),
        in_specs=[
            pl.BlockSpec((4, BN, 128), lambda i: (0, i, 0)),
            pl.BlockSpec((1, 1, BN), lambda i: (i, 0, 0)),
            full((4, 8, 128)),
            full((3 * GRU_H, H3)),
            full((8, 3 * GRU_H)),
            full((8, 3 * GRU_H)),
            full((3 * GRU_H, GRU_H)),
            full((8, 3 * GRU_H)),
            full((8, 3 * GRU_H)),
            full((H3, GRU_H)),
            full((8, H3)),
        ],
        out_specs=pl.BlockSpec((N_GRAPHS, H3), lambda i: (0, 0)),
        out_shape=jax.ShapeDtypeStruct((N_GRAPHS, H3), jnp.float32),
        scratch_shapes=[pltpu.VMEM((N_GRAPHS, H3), jnp.float32),
                        pltpu.VMEM((N_GRAPHS, 128), jnp.float32)],
        interpret=_INTERPRET,
    )(hc3, bidx3d, b3c, W_ih1, b_ih1, b_hh1, W_ih2, b_ih2, b_hh2, Wo, bo)


def _chunk_vecs(a_src, a_dst, c_out):
    asrc_c = a_src.reshape(c_out, 128)
    adst_c = a_dst.reshape(c_out, 128)
    av = jnp.stack([asrc_c, adst_c])
    return jnp.broadcast_to(av[:, :, None, :], (2, c_out, 8, 128))


def _bias_chunks(b, c):
    return jnp.broadcast_to(b.reshape(c, 1, 128), (c, 8, 128))


def _layer(a, src, dst, W, a_src, a_dst, b_prev, relu_in, heads, out_dim):
    c_in = a.shape[0]
    c_out = heads * out_dim // 128
    dpc = out_dim // 128
    w_r = W.reshape(c_in, 128, heads * out_dim)
    bias_c = (_bias_chunks(b_prev, c_in) if relu_in
              else jnp.zeros((c_in, 8, 128), jnp.float32))
    avec = _chunk_vecs(a_src, a_dst, c_out)
    hc, al, m_arr = _stage_a(a, w_r, bias_c, avec, relu_in, dpc)
    w_att = _edge_softmax(al, m_arr, src, dst)
    return _aggregate(hc, w_att, src, dst, dpc)


def kernel(x, edge_index, batch_idx, W1, a_src1, a_dst1, b1, W2, a_src2,
           a_dst2, b2, W3, a_src3, a_dst3, b3, W_ih1, W_hh1, b_ih1, b_hh1,
           W_ih2, W_hh2, b_ih2, b_hh2, Wo, bo):
    src, dst = edge_index[0], edge_index[1]
    xp = jnp.pad(x, ((0, NP - N_NODES), (0, 128 - x.shape[1])))[None]
    w1p = jnp.pad(W1, ((0, 128 - W1.shape[0]), (0, 0)))

    g1 = _layer(xp, src, dst, w1p, a_src1, a_dst1, None, False, HEADS, H1)
    g2 = _layer(g1, src, dst, W2, a_src2, a_dst2, b1, True, HEADS, H2)
    g3 = _layer(g2, src, dst, W3, a_src3, a_dst3, b2, True, 1, H3)

    bidx3d = jnp.pad(batch_idx, (0, NP - N_NODES),
                     constant_values=N_GRAPHS).reshape(NT, 1, BN)
    b3c = _bias_chunks(b3, 4)
    b8 = lambda v: jnp.broadcast_to(v[None, :], (8, v.shape[0]))
    return _stage_d(g3, bidx3d, b3c, W_ih1, b8(b_ih1), b8(b_hh1), W_ih2,
                    b8(b_ih2), b8(b_hh2), Wo, b8(bo))
